# baseline (speedup 1.0000x reference)
import functools

import jax
import jax.numpy as jnp
from jax.experimental import pallas as pl
from jax.experimental.pallas import tpu as pltpu

_INV_TEMP = 10.0
_EPS_SQ = 1e-24


def _smog_logits_kernel(x_ref, g_ref, out_ref, gs_ref):
    @pl.when(pl.program_id(0) == 0)
    def _():
        g = g_ref[...]
        gs_ref[...] = g * jax.lax.rsqrt(
            jnp.maximum(jnp.sum(g * g, axis=1, keepdims=True), _EPS_SQ))

    x = x_ref[...]
    xs = x * (_INV_TEMP * jax.lax.rsqrt(
        jnp.maximum(jnp.sum(x * x, axis=1, keepdims=True), _EPS_SQ)))
    out_ref[...] = jax.lax.dot_general(
        xs, gs_ref[...], (((1,), (1,)), ((), ())),
        preferred_element_type=jnp.float32)


@functools.partial(jax.jit, static_argnames=("bm",))
def _smog_logits(x, group_features, bm):
    b, d = x.shape
    k, _ = group_features.shape
    bm = min(bm, b)
    return pl.pallas_call(
        _smog_logits_kernel,
        grid=(b // bm,),
        in_specs=[
            pl.BlockSpec((bm, d), lambda i: (i, 0)),
            pl.BlockSpec((k, d), lambda i: (0, 0)),
        ],
        out_specs=pl.BlockSpec((bm, k), lambda i: (i, 0)),
        out_shape=jax.ShapeDtypeStruct((b, k), jnp.float32),
        scratch_shapes=[pltpu.VMEM((k, d), jnp.float32)],
        compiler_params=pltpu.CompilerParams(
            dimension_semantics=("arbitrary",)),
    )(x, group_features)


def kernel(x, group_features):
    return _smog_logits(x, group_features, bm=128)


# speedup vs baseline: 1.0988x; 1.0988x over previous
import functools

import jax
import jax.numpy as jnp
from jax.experimental import pallas as pl
from jax.experimental.pallas import tpu as pltpu

_INV_TEMP = 10.0
_EPS_SQ = 1e-24
_NBUF = 4


def _smog_kernel(x_ref, g_ref, out_ref, buf_ref, sem_ref, *, bm):
    i = pl.program_id(0)
    nsteps = pl.num_programs(0)
    slot = jax.lax.rem(i, _NBUF)

    @pl.when(i >= _NBUF)
    def _():
        pltpu.make_async_copy(
            buf_ref.at[slot],
            out_ref.at[pl.ds((i - _NBUF) * bm, bm), :],
            sem_ref.at[slot]).wait()

    x = x_ref[...]
    g = g_ref[...]
    xs = x * (_INV_TEMP * jax.lax.rsqrt(
        jnp.maximum(jnp.sum(x * x, axis=1, keepdims=True), _EPS_SQ)))
    gs = g * jax.lax.rsqrt(
        jnp.maximum(jnp.sum(g * g, axis=1, keepdims=True), _EPS_SQ))
    buf_ref[slot] = jax.lax.dot_general(
        xs, gs, (((1,), (1,)), ((), ())),
        preferred_element_type=jnp.float32)
    pltpu.make_async_copy(
        buf_ref.at[slot],
        out_ref.at[pl.ds(i * bm, bm), :],
        sem_ref.at[slot]).start()

    @pl.when(i == nsteps - 1)
    def _():
        def drain(j, _):
            s = jax.lax.rem(i - (_NBUF - 1) + j, _NBUF)
            pltpu.make_async_copy(
                buf_ref.at[s],
                out_ref.at[pl.ds((i - (_NBUF - 1) + j) * bm, bm), :],
                sem_ref.at[s]).wait()
            return 0
        jax.lax.fori_loop(0, _NBUF, drain, 0)


@functools.partial(jax.jit, static_argnames=("bm",))
def _smog_logits(x, group_features, bm):
    b, d = x.shape
    k, _ = group_features.shape
    bm = min(bm, b)
    return pl.pallas_call(
        functools.partial(_smog_kernel, bm=bm),
        grid=(b // bm,),
        in_specs=[
            pl.BlockSpec((bm, d), lambda i: (i, 0)),
            pl.BlockSpec((k, d), lambda i: (0, 0)),
        ],
        out_specs=pl.BlockSpec(memory_space=pl.ANY),
        out_shape=jax.ShapeDtypeStruct((b, k), jnp.float32),
        scratch_shapes=[
            pltpu.VMEM((_NBUF, bm, k), jnp.float32),
            pltpu.SemaphoreType.DMA((_NBUF,)),
        ],
        compiler_params=pltpu.CompilerParams(
            dimension_semantics=("arbitrary",)),
    )(x, group_features)


def kernel(x, group_features):
    return _smog_logits(x, group_features, bm=256)
